# baseline (speedup 1.0000x reference)
import functools

import jax
import jax.numpy as jnp
from jax import lax
from jax.experimental import pallas as pl
from jax.experimental.pallas import tpu as pltpu
from jax.experimental.pallas import tpu_sc as plsc

NC = 2
NS = 16
NW = NC * NS
RPC = 128
NBUF = 4
GAHEAD = 2
UN = 8
LANES = 16


def _pos_add(rows_ref, pos2_ref, p0, dim):
  def body(i, carry):
    r0 = i * UN
    for u in range(UN):
      r = r0 + u
      pr = p0 + r
      for h in range(dim // LANES):
        v = pos2_ref[pr, pl.ds(h * LANES, LANES)]
        plsc.addupdate(rows_ref.at[r, pl.ds(h * LANES, LANES)], v)
    return carry

  lax.fori_loop(0, RPC // UN, body, 0)


@functools.lru_cache(maxsize=None)
def _build(nbatch, seqlen, dim):
  rows_per_w = nbatch * seqlen // NW
  ch = rows_per_w // RPC
  assert nbatch * seqlen == NW * rows_per_w
  assert rows_per_w % RPC == 0 and rows_per_w % seqlen == 0
  assert (ch - 2 * GAHEAD) % NBUF == 0
  assert dim % LANES == 0

  mesh = plsc.VectorSubcoreMesh(core_axis_name="c", subcore_axis_name="s")

  @functools.partial(
      pl.kernel,
      mesh=mesh,
      out_type=jax.ShapeDtypeStruct((NW, ch, RPC, dim), jnp.float32),
      scratch_types=[
          pltpu.VMEM((ch, RPC), jnp.int32),
          pltpu.VMEM((2 * seqlen, dim), jnp.float32),
          pltpu.VMEM((NBUF, RPC, dim), jnp.float32),
      ] + [pltpu.SemaphoreType.DMA] * (2 * NBUF),
      compiler_params=pltpu.CompilerParams(use_tc_tiling_on_sc=False),
  )
  def k(ids_hbm, tok_hbm, pos_hbm, out_hbm, idx_v, pos2_v, rows_v, *sems):
    gs = sems[:NBUF]
    ws = sems[NBUF:]
    wid = lax.axis_index("s") * NC + lax.axis_index("c")

    def gstart(j, b):
      pltpu.make_async_copy(
          tok_hbm.at[idx_v.at[j]], rows_v.at[b], gs[b]).start()

    def gwait(b):
      pltpu.make_async_copy(
          tok_hbm.at[idx_v.at[0]], rows_v.at[b], gs[b]).wait()

    def wstart(j, b):
      pltpu.make_async_copy(rows_v.at[b], out_hbm.at[wid, j], ws[b]).start()

    def wwait(b):
      pltpu.make_async_copy(rows_v.at[b], out_hbm.at[wid, 0], ws[b]).wait()

    pltpu.sync_copy(ids_hbm.at[wid], idx_v)
    pltpu.sync_copy(pos_hbm, pos2_v.at[pl.ds(0, seqlen)])
    pltpu.sync_copy(pos_hbm, pos2_v.at[pl.ds(seqlen, seqlen)])

    for j in range(GAHEAD):
      gstart(j, j % NBUF)

    for j in range(GAHEAD):
      gstart(j + GAHEAD, (j + GAHEAD) % NBUF)
      gwait(j % NBUF)
      _pos_add(rows_v.at[j % NBUF], pos2_v, (j * RPC) % seqlen, dim)
      wstart(j, j % NBUF)

    def outer(i, carry):
      for b in range(NBUF):
        j = GAHEAD + i * NBUF + b
        bcur = (GAHEAD + b) % NBUF
        bnext = b % NBUF
        wwait(bnext)
        gstart(j + GAHEAD, bnext)
        gwait(bcur)
        p0 = lax.rem(j * RPC, seqlen)
        _pos_add(rows_v.at[bcur], pos2_v, p0, dim)
        wstart(j, bcur)
      return carry

    lax.fori_loop(0, (ch - 2 * GAHEAD) // NBUF, outer, 0)

    for j in range(ch - GAHEAD, ch):
      b = j % NBUF
      gwait(b)
      _pos_add(rows_v.at[b], pos2_v, (j * RPC) % seqlen, dim)
      wstart(j, b)

    for b in range(NBUF):
      wwait(b)

  return k


def kernel(input_ids, token_embedding, position_embedding):
  nbatch, seqlen = input_ids.shape
  dim = token_embedding.shape[-1]
  ids3 = input_ids.astype(jnp.int32).reshape(NW, -1, RPC)
  out = _build(nbatch, seqlen, dim)(
      ids3, token_embedding, position_embedding)
  return out.reshape(nbatch, seqlen, dim)


# speedup vs baseline: 1.4758x; 1.4758x over previous
import functools

import jax
import jax.numpy as jnp
from jax import lax
from jax.experimental import pallas as pl
from jax.experimental.pallas import tpu as pltpu
from jax.experimental.pallas import tpu_sc as plsc

NC = 2
NS = 16
NW = NC * NS
RPC = 128
PADW = 128
NBG = 3
NBW = 2
HEAD = 6
TAIL = 2
LANES = 16


def _add_compact(rows_ref, cbuf_ref, pos2c_ref, qbase, dim):
  nh = dim // LANES

  @plsc.parallel_loop(0, RPC // 2, step=1, unroll=4)
  def _(r2):
    q = qbase + r2
    for u in range(2):
      r = 2 * r2 + u
      for h in range(nh):
        v = (rows_ref[r, pl.ds(h * LANES, LANES)]
             + pos2c_ref[q, pl.ds(u * dim + h * LANES, LANES)])
        cbuf_ref[r, pl.ds(h * LANES, LANES)] = v


@functools.lru_cache(maxsize=None)
def _build(nbatch, seqlen, dim):
  rows_per_w = nbatch * seqlen // NW
  ch = rows_per_w // RPC
  assert nbatch * seqlen == NW * rows_per_w
  assert rows_per_w % RPC == 0 and rows_per_w % seqlen == 0
  assert (ch - HEAD - TAIL) % (NBG * NBW) == 0
  assert dim % LANES == 0 and 2 * dim == PADW

  p2rows = seqlen + RPC
  p2packed = (p2rows + 1) // 2

  mesh = plsc.VectorSubcoreMesh(core_axis_name="c", subcore_axis_name="s")

  @functools.partial(
      pl.kernel,
      mesh=mesh,
      out_type=jax.ShapeDtypeStruct((NW, ch, RPC, dim), jnp.float32),
      scratch_types=[
          pltpu.VMEM((ch, RPC), jnp.int32),
          pltpu.VMEM((p2packed, 2 * dim), jnp.float32),
          pltpu.VMEM((NBG, RPC, PADW), jnp.float32),
          pltpu.VMEM((NBW, RPC, dim), jnp.float32),
      ] + [pltpu.SemaphoreType.DMA] * (NBG + NBW),
      compiler_params=pltpu.CompilerParams(use_tc_tiling_on_sc=True),
  )
  def k(ids_hbm, tok_hbm, pos2c_hbm, out_hbm,
        idx_v, pos2c_v, rows_v, cbuf_v, *sems):
    gs = sems[:NBG]
    ws = sems[NBG:]
    wid = lax.axis_index("s") * NC + lax.axis_index("c")

    def gstart(j, g):
      pltpu.make_async_copy(
          tok_hbm.at[idx_v.at[j]], rows_v.at[g], gs[g]).start()

    def gwait(g):
      pltpu.make_async_copy(
          tok_hbm.at[idx_v.at[0]], rows_v.at[g], gs[g]).wait()

    def wstart(j, w):
      pltpu.make_async_copy(cbuf_v.at[w], out_hbm.at[wid, j], ws[w]).start()

    def wwait(w):
      pltpu.make_async_copy(cbuf_v.at[w], out_hbm.at[wid, 0], ws[w]).wait()

    pltpu.sync_copy(ids_hbm.at[wid], idx_v)
    pltpu.sync_copy(pos2c_hbm, pos2c_v)

    def chunk(j, p0, do_wwait, issue_j, g, w):
      if do_wwait:
        wwait(w)
      if issue_j is not None:
        gstart(issue_j, (g + 2) % NBG)
      gwait(g)
      _add_compact(rows_v.at[g], cbuf_v.at[w], pos2c_v, p0 // 2, dim)
      wstart(j, w)

    for j in range(2):
      gstart(j, j % NBG)

    for j in range(HEAD):
      chunk(j, (j * RPC) % seqlen, j >= NBW, j + 2, j % NBG, j % NBW)

    def outer(i, carry):
      for b in range(NBG * NBW):
        j = HEAD + i * (NBG * NBW) + b
        p0 = lax.rem(j * RPC, seqlen)
        chunk(j, p0, True, j + 2, (HEAD + b) % NBG, (HEAD + b) % NBW)
      return carry

    lax.fori_loop(0, (ch - HEAD - TAIL) // (NBG * NBW), outer, 0)

    for j in range(ch - TAIL, ch):
      chunk(j, (j * RPC) % seqlen, True, None, j % NBG, j % NBW)

    for w in range(NBW):
      wwait(w)

  return k


def kernel(input_ids, token_embedding, position_embedding):
  nbatch, seqlen = input_ids.shape
  dim = token_embedding.shape[-1]
  ids3 = input_ids.astype(jnp.int32).reshape(NW, -1, RPC)
  tok_pad = jnp.pad(token_embedding, ((0, 0), (0, PADW - dim)))
  p2rows = seqlen + RPC
  p2packed = (p2rows + 1) // 2
  pos2c = jnp.concatenate(
      [position_embedding, position_embedding], axis=0)[:2 * p2packed]
  pos2c = pos2c.reshape(p2packed, 2 * dim)
  out = _build(nbatch, seqlen, dim)(ids3, tok_pad, pos2c)
  return out.reshape(nbatch, seqlen, dim)


# speedup vs baseline: 1.4761x; 1.0002x over previous
import functools

import jax
import jax.numpy as jnp
from jax import lax
from jax.experimental import pallas as pl
from jax.experimental.pallas import tpu as pltpu
from jax.experimental.pallas import tpu_sc as plsc

NC = 2
NS = 16
NW = NC * NS
RPC = 128
PADW = 128
NBG = 4
NBW = 2
NBI = 4
HEAD = 4
TAIL = 4
LANES = 16


def _add_compact(rows_ref, cbuf_ref, pos2c_ref, qbase, dim):
  nh = dim // LANES

  @plsc.parallel_loop(0, RPC // 2, step=1, unroll=4)
  def _(r2):
    q = qbase + r2
    for u in range(2):
      r = 2 * r2 + u
      for h in range(nh):
        v = (rows_ref[r, pl.ds(h * LANES, LANES)]
             + pos2c_ref[q, pl.ds(u * dim + h * LANES, LANES)])
        cbuf_ref[r, pl.ds(h * LANES, LANES)] = v


@functools.lru_cache(maxsize=None)
def _build(nbatch, seqlen, dim):
  rows_per_w = nbatch * seqlen // NW
  ch = rows_per_w // RPC
  assert nbatch * seqlen == NW * rows_per_w
  assert rows_per_w % RPC == 0 and rows_per_w % seqlen == 0
  assert (ch - HEAD - TAIL) % NBG == 0 and NBG % NBW == 0
  assert dim % LANES == 0 and 2 * dim == PADW

  p2rows = seqlen + RPC
  p2packed = (p2rows + 1) // 2

  mesh = plsc.VectorSubcoreMesh(core_axis_name="c", subcore_axis_name="s")

  @functools.partial(
      pl.kernel,
      mesh=mesh,
      out_type=jax.ShapeDtypeStruct((NW, ch, RPC, dim), jnp.float32),
      scratch_types=[
          pltpu.VMEM((NBI, RPC), jnp.int32),
          pltpu.VMEM((p2packed, 2 * dim), jnp.float32),
          pltpu.VMEM((NBG, RPC, PADW), jnp.float32),
          pltpu.VMEM((NBW, RPC, dim), jnp.float32),
      ] + [pltpu.SemaphoreType.DMA] * (NBI + NBG + NBW),
      compiler_params=pltpu.CompilerParams(use_tc_tiling_on_sc=True),
  )
  def k(ids_hbm, tok_hbm, pos2c_hbm, out_hbm,
        idxr_v, pos2c_v, rows_v, cbuf_v, *sems):
    isems = sems[:NBI]
    gs = sems[NBI:NBI + NBG]
    ws = sems[NBI + NBG:]
    wid = lax.axis_index("s") * NC + lax.axis_index("c")

    def istart(j, s):
      pltpu.make_async_copy(ids_hbm.at[wid, j], idxr_v.at[s], isems[s]).start()

    def iwait(s):
      pltpu.make_async_copy(
          ids_hbm.at[wid, 0], idxr_v.at[s], isems[s]).wait()

    def gstart(s, g):
      pltpu.make_async_copy(
          tok_hbm.at[idxr_v.at[s]], rows_v.at[g], gs[g]).start()

    def gwait(g):
      pltpu.make_async_copy(
          tok_hbm.at[idxr_v.at[0]], rows_v.at[g], gs[g]).wait()

    def wstart(j, w):
      pltpu.make_async_copy(cbuf_v.at[w], out_hbm.at[wid, j], ws[w]).start()

    def wwait(w):
      pltpu.make_async_copy(cbuf_v.at[w], out_hbm.at[wid, 0], ws[w]).wait()

    pltpu.sync_copy(pos2c_hbm, pos2c_v)

    def chunk(j, p0, g, w, do_wwait=True, issue_g=True, issue_i=True):
      if do_wwait:
        wwait(w)
      if issue_g:
        sl = (g + 3) % NBI
        iwait(sl)
        gstart(sl, (g + 3) % NBG)
      gwait(g)
      if issue_i:
        istart(j + 4, g % NBI)
      _add_compact(rows_v.at[g], cbuf_v.at[w], pos2c_v, p0 // 2, dim)
      wstart(j, w)

    for j in range(NBI):
      istart(j, j)
    for j in range(3):
      iwait(j)
      gstart(j, j)

    for j in range(HEAD):
      chunk(j, (j * RPC) % seqlen, j % NBG, j % NBW, do_wwait=j >= NBW)

    def outer(i, carry):
      for b in range(NBG):
        j = HEAD + i * NBG + b
        p0 = lax.rem(j * RPC, seqlen)
        chunk(j, p0, b % NBG, b % NBW)
      return carry

    lax.fori_loop(0, (ch - HEAD - TAIL) // NBG, outer, 0)

    for j in range(ch - TAIL, ch):
      chunk(j, (j * RPC) % seqlen, j % NBG, j % NBW,
            issue_g=(j + 3 < ch), issue_i=False)

    for w in range(NBW):
      wwait(w)

  return k


def kernel(input_ids, token_embedding, position_embedding):
  nbatch, seqlen = input_ids.shape
  dim = token_embedding.shape[-1]
  ids3 = input_ids.astype(jnp.int32).reshape(NW, -1, RPC)
  tok_pad = jnp.pad(token_embedding, ((0, 0), (0, PADW - dim)))
  p2rows = seqlen + RPC
  p2packed = (p2rows + 1) // 2
  pos2c = jnp.concatenate(
      [position_embedding, position_embedding], axis=0)[:2 * p2packed]
  pos2c = pos2c.reshape(p2packed, 2 * dim)
  out = _build(nbatch, seqlen, dim)(ids3, tok_pad, pos2c)
  return out.reshape(nbatch, seqlen, dim)
